# Initial kernel scaffold; baseline (speedup 1.0000x reference)
import jax
import jax.numpy as jnp
from jax.experimental import pallas as pl


def kernel(x, table, W1, b1, W2, b2):
    raise NotImplementedError("write your pallas kernel here")


# speedup vs baseline: 30.6112x; 30.6112x over previous
import functools

import jax
import jax.numpy as jnp
from jax import lax
from jax.experimental import pallas as pl
from jax.experimental.pallas import tpu as pltpu
from jax.experimental.pallas import tpu_sc as plsc

_B = 16384
_L = 200
_EMB = 10
_H1 = 16
_V = 100001

_NW = 32
_RPW = _B // _NW
_C = 16
_NCHUNK = _RPW // _C
_HL = _L // 2

_mesh = plsc.VectorSubcoreMesh(core_axis_name="c", subcore_axis_name="s")


def _t1_body(tab_ref, w_ref, o_ref):
    o_ref[...] = jnp.dot(
        tab_ref[...], w_ref[...], preferred_element_type=jnp.float32
    ) * (1.0 / _L)


def _make_t1(table, w1):
    blk = 8192
    grid = (_V + blk - 1) // blk
    return pl.pallas_call(
        _t1_body,
        grid=(grid,),
        in_specs=[
            pl.BlockSpec((blk, _EMB), lambda i: (i, 0)),
            pl.BlockSpec((_EMB, _H1), lambda i: (0, 0)),
        ],
        out_specs=pl.BlockSpec((blk, _H1), lambda i: (i, 0)),
        out_shape=jax.ShapeDtypeStruct((_V, _H1), jnp.float32),
    )(table, w1)


@functools.partial(
    pl.kernel,
    mesh=_mesh,
    compiler_params=pltpu.CompilerParams(
        needs_layout_passes=False, use_tc_tiling_on_sc=False),
    out_type=jax.ShapeDtypeStruct((_B,), jnp.float32),
    scratch_types=[
        pltpu.VMEM((_C * 2, _HL), jnp.int32),
        pltpu.VMEM((_C * _L, _H1), jnp.float32),
        pltpu.VMEM((_C * 16,), jnp.float32),
        pltpu.VMEM((_C,), jnp.float32),
        pltpu.VMEM((3, 16), jnp.float32),
        pltpu.SemaphoreType.DMA,
    ],
)
def _sc_pool(x_hbm, t1_hbm, consts_hbm, out_hbm,
             idx_v, rows_v, tr_v, outb_v, c_v, sem):
    wid = lax.axis_index("s") * 2 + lax.axis_index("c")
    base = wid * _RPW
    pltpu.sync_copy(consts_hbm, c_v)
    b1v = c_v[0, :]
    w2v = c_v[1, :]
    b2v = c_v[2, :]

    def chunk_body(ch, carry):
        rowbase = base + ch * _C
        pltpu.sync_copy(x_hbm.at[pl.ds(rowbase * 2, _C * 2)], idx_v)
        descs = [
            pltpu.async_copy(t1_hbm.at[idx_v.at[h]],
                             rows_v.at[pl.ds(h * _HL, _HL)], sem)
            for h in range(_C * 2)
        ]
        for d in descs:
            d.wait()
        lanes = lax.iota(jnp.int32, 16)
        for r in range(_C):
            def j_body(t, acc, _r=r):
                row = jnp.full((16,), _r * _L, jnp.int32) + t
                return acc + plsc.load_gather(rows_v, [row, lanes])
            acc = lax.fori_loop(0, _L, j_body,
                                jnp.zeros((16,), jnp.float32), unroll=8)
            plsc.store_scatter(tr_v, [lanes * _C + r], acc)
        z = jnp.zeros((_C,), jnp.float32)
        for lane in range(_H1):
            col = tr_v[pl.ds(lane * _C, _C)]
            h = jnp.maximum(col + b1v[lane], 0.0)
            z = z + h * w2v[lane]
        z = z + b2v[0]
        outb_v[...] = 1.0 / (1.0 + jnp.exp(-z))
        pltpu.sync_copy(outb_v, out_hbm.at[pl.ds(rowbase, _C)])
        return carry

    lax.fori_loop(0, _NCHUNK, chunk_body, 0)


def kernel(x, table, W1, b1, W2, b2):
    t1 = _make_t1(table, W1)
    consts = jnp.concatenate(
        [b1[None, :], W2.T, jnp.full((1, _H1), b2[0], jnp.float32)], axis=0)
    x2 = x.astype(jnp.int32).reshape(_B * 2, _HL)
    out = _sc_pool(x2, t1, consts)
    return out.reshape(_B, 1)


# speedup vs baseline: 30.8725x; 1.0085x over previous
import functools

import jax
import jax.numpy as jnp
from jax import lax
from jax.experimental import pallas as pl
from jax.experimental.pallas import tpu as pltpu
from jax.experimental.pallas import tpu_sc as plsc

_B = 16384
_L = 200
_EMB = 10
_H1 = 16
_V = 100001

_NW = 32
_RPW = _B // _NW
_C = 16
_NCHUNK = _RPW // _C
_HL = _L // 2

_mesh = plsc.VectorSubcoreMesh(core_axis_name="c", subcore_axis_name="s")


def _t1_body(tab_ref, w_ref, o_ref):
    o_ref[...] = jnp.dot(
        tab_ref[...], w_ref[...], preferred_element_type=jnp.float32
    ) * (1.0 / _L)


def _make_t1(table, w1):
    blk = 8192
    grid = (_V + blk - 1) // blk
    return pl.pallas_call(
        _t1_body,
        grid=(grid,),
        in_specs=[
            pl.BlockSpec((blk, _EMB), lambda i: (i, 0)),
            pl.BlockSpec((_EMB, _H1), lambda i: (0, 0)),
        ],
        out_specs=pl.BlockSpec((blk, _H1), lambda i: (i, 0)),
        out_shape=jax.ShapeDtypeStruct((_V, _H1), jnp.float32),
    )(table, w1)


@functools.partial(
    pl.kernel,
    mesh=_mesh,
    compiler_params=pltpu.CompilerParams(
        needs_layout_passes=False, use_tc_tiling_on_sc=False),
    out_type=jax.ShapeDtypeStruct((_B,), jnp.float32),
    scratch_types=[
        pltpu.VMEM((_C * 2, _HL), jnp.int32),
        pltpu.VMEM((_C * _L, _H1), jnp.float32),
        pltpu.VMEM((_C * 16,), jnp.float32),
        pltpu.VMEM((_C,), jnp.float32),
        pltpu.VMEM((3, 16), jnp.float32),
        pltpu.SemaphoreType.DMA,
    ],
)
def _sc_pool(x_hbm, t1_hbm, consts_hbm, out_hbm,
             idx_v, rows_v, tr_v, outb_v, c_v, sem):
    wid = lax.axis_index("s") * 2 + lax.axis_index("c")
    base = wid * _RPW
    pltpu.sync_copy(consts_hbm, c_v)
    b1v = c_v[0, :]
    w2v = c_v[1, :]
    b2v = c_v[2, :]

    def chunk_body(ch, carry):
        rowbase = base + ch * _C
        pltpu.sync_copy(x_hbm.at[pl.ds(rowbase * 2, _C * 2)], idx_v)
        descs = [
            pltpu.async_copy(t1_hbm.at[idx_v.at[h]],
                             rows_v.at[pl.ds(h * _HL, _HL)], sem)
            for h in range(_C * 2)
        ]
        for d in descs:
            d.wait()
        lanes = lax.iota(jnp.int32, 16)
        for r in range(_C):
            def j_body(t, acc, _r=r):
                return acc + rows_v[_r * _L + t, :]
            acc = lax.fori_loop(0, _L, j_body,
                                jnp.zeros((16,), jnp.float32), unroll=8)
            plsc.store_scatter(tr_v, [lanes * _C + r], acc)
        z = jnp.zeros((_C,), jnp.float32)
        for lane in range(_H1):
            col = tr_v[pl.ds(lane * _C, _C)]
            h = jnp.maximum(col + b1v[lane], 0.0)
            z = z + h * w2v[lane]
        z = z + b2v[0]
        outb_v[...] = 1.0 / (1.0 + jnp.exp(-z))
        pltpu.sync_copy(outb_v, out_hbm.at[pl.ds(rowbase, _C)])
        return carry

    lax.fori_loop(0, _NCHUNK, chunk_body, 0)


def kernel(x, table, W1, b1, W2, b2):
    t1 = _make_t1(table, W1)
    consts = jnp.concatenate(
        [b1[None, :], W2.T, jnp.full((1, _H1), b2[0], jnp.float32)], axis=0)
    x2 = x.astype(jnp.int32).reshape(_B * 2, _HL)
    out = _sc_pool(x2, t1, consts)
    return out.reshape(_B, 1)
